# baseline (speedup 1.0000x reference)
import functools

import jax
import jax.numpy as jnp
from jax import lax
from jax.experimental import pallas as pl
from jax.experimental.pallas import tpu as pltpu
from jax.experimental.pallas import tpu_sc as plsc

B, T, D = 2, 2048, 1024
E, TOPK, H = 8, 2, 512
SH = H * TOPK
N = B * T
M = N * TOPK
TM2 = 128
BUF = M + E * TM2
NTILES = BUF // TM2
NW = 32
RPW = N // NW
TMS = 1024


def _cumsum_rows(a):
    rows = lax.broadcasted_iota(jnp.int32, a.shape, 0)
    k = 1
    while k < a.shape[0]:
        shifted = pltpu.roll(a, k, axis=0)
        a = a + jnp.where(rows >= k, shifted, jnp.zeros_like(a))
        k *= 2
    return a


def _router_body(x_ref, rw_ref, rb_ref, pos_ref, w_ref, off_ref):
    x = x_ref[...]
    logits = jnp.dot(x, rw_ref[...].T, preferred_element_type=jnp.float32)
    logits = logits + rb_ref[...]
    scores = jax.nn.softmax(logits, axis=-1)
    s1 = jnp.max(scores, axis=-1, keepdims=True)
    i1 = jnp.argmax(scores, axis=-1).reshape(N, 1)
    cols = lax.broadcasted_iota(jnp.int32, (N, E), 1)
    masked = jnp.where(cols == i1, -jnp.inf, scores)
    s2 = jnp.max(masked, axis=-1, keepdims=True)
    i2 = jnp.argmax(masked, axis=-1).reshape(N, 1)
    denom = s1 + s2
    w1 = s1 / denom
    w2 = s2 / denom

    oh1 = (cols == i1).astype(jnp.int32)
    oh2 = (cols == i2).astype(jnp.int32)
    packed = _cumsum_rows(oh1 + (oh2 << 13))
    cs1 = packed & 8191
    cs2 = packed >> 13
    cnt1 = cs1[N - 1:N, :]
    counts = cnt1 + cs2[N - 1:N, :]
    padded = (counts + (TM2 - 1)) & (-TM2)
    padded_f = padded.astype(jnp.float32)
    r8 = lax.broadcasted_iota(jnp.int32, (E, E), 0)
    c8 = lax.broadcasted_iota(jnp.int32, (E, E), 1)
    tril = (r8 < c8).astype(jnp.float32)
    offf = jnp.dot(padded_f, tril, preferred_element_type=jnp.float32)
    off = offf.astype(jnp.int32)

    rk1 = jnp.sum(oh1 * cs1, axis=1, keepdims=True) - 1
    rk2 = jnp.sum(oh2 * (cs2 + cnt1), axis=1, keepdims=True) - 1
    base1 = jnp.sum(oh1 * off, axis=1, keepdims=True)
    base2 = jnp.sum(oh2 * off, axis=1, keepdims=True)
    pos1 = base1 + rk1
    pos2 = base2 + rk2

    c128 = lax.broadcasted_iota(jnp.int32, (N, 128), 1)
    pos_ref[...] = jnp.where(c128 == 0, pos1, jnp.where(c128 == 1, pos2, 0))
    w_ref[...] = jnp.where(c128 == 0, w1, jnp.where(c128 == 1, w2, 0.0))
    spread = (lax.broadcasted_iota(jnp.int32, (E, 128), 0)
              == lax.broadcasted_iota(jnp.int32, (E, 128), 1)).astype(jnp.float32)
    off_ref[...] = jnp.dot(offf, spread,
                           preferred_element_type=jnp.float32).astype(jnp.int32)


def _router_meta(flat, router_w, rb):
    return pl.pallas_call(
        _router_body,
        out_shape=[
            jax.ShapeDtypeStruct((N, 128), jnp.int32),
            jax.ShapeDtypeStruct((N, 128), jnp.float32),
            jax.ShapeDtypeStruct((1, 128), jnp.int32),
        ],
    )(flat, router_w, rb)


def _shared_body(x_ref, sg_ref, su_ref, sd_ref, out_ref):
    x = x_ref[...]
    g = jnp.dot(x, sg_ref[...].T, preferred_element_type=jnp.float32)
    u = jnp.dot(x, su_ref[...].T, preferred_element_type=jnp.float32)
    h = (g * jax.nn.sigmoid(g)) * u
    out_ref[...] = jnp.dot(h, sd_ref[...].T, preferred_element_type=jnp.float32)


def _shared_ffn(flat, sg_w, su_w, sd_w):
    return pl.pallas_call(
        _shared_body,
        grid=(N // TMS,),
        in_specs=[
            pl.BlockSpec((TMS, D), lambda t: (t, 0)),
            pl.BlockSpec((SH, D), lambda t: (0, 0)),
            pl.BlockSpec((SH, D), lambda t: (0, 0)),
            pl.BlockSpec((D, SH), lambda t: (0, 0)),
        ],
        out_specs=pl.BlockSpec((TMS, D), lambda t: (t, 0)),
        out_shape=jax.ShapeDtypeStruct((N, D), jnp.float32),
        compiler_params=pltpu.CompilerParams(
            dimension_semantics=("parallel",)),
    )(flat, sg_w, su_w, sd_w)


def _dispatch_body(xb_hbm, posq_hbm, gx_hbm, rows_v, idx0_v, idx1_v, sem):
    cid = lax.axis_index("c")
    sid = lax.axis_index("s")
    wid = sid * 2 + cid
    for c in range(2):
        base = wid * RPW + c * 64
        pltpu.sync_copy(xb_hbm.at[pl.ds(base, 64)], rows_v)
        pltpu.sync_copy(posq_hbm.at[wid, c, 0], idx0_v)
        pltpu.sync_copy(posq_hbm.at[wid, c, 1], idx1_v)
        d0 = pltpu.async_copy(rows_v, gx_hbm.at[idx0_v], sem)
        d1 = pltpu.async_copy(rows_v, gx_hbm.at[idx1_v], sem)
        d0.wait()
        d1.wait()


def _dispatch(xb32, posq):
    mesh = plsc.VectorSubcoreMesh(core_axis_name="c", subcore_axis_name="s")
    return pl.kernel(
        _dispatch_body,
        out_type=jax.ShapeDtypeStruct((BUF, D // 2), jnp.int32),
        mesh=mesh,
        scratch_types=[
            pltpu.VMEM((64, D // 2), jnp.int32),
            pltpu.VMEM((64,), jnp.int32),
            pltpu.VMEM((64,), jnp.int32),
            pltpu.SemaphoreType.DMA,
        ],
    )(xb32, posq)


def _group_gemm_body(te_ref, gx_ref, gw_ref, uw_ref, dw_ref, y_ref):
    xg = gx_ref[...]
    g = jnp.dot(xg, gw_ref[0].T, preferred_element_type=jnp.float32)
    u = jnp.dot(xg, uw_ref[0].T, preferred_element_type=jnp.float32)
    h = (g * jax.nn.sigmoid(g)) * u
    p = jnp.dot(h, dw_ref[0].T, preferred_element_type=jnp.float32)
    y_ref[...] = p.astype(jnp.bfloat16)


def _group_gemm(te, gx, gate_w, up_w, down_w):
    grid_spec = pltpu.PrefetchScalarGridSpec(
        num_scalar_prefetch=1,
        grid=(NTILES,),
        in_specs=[
            pl.BlockSpec((TM2, D), lambda i, te: (i, 0)),
            pl.BlockSpec((1, H, D), lambda i, te: (te[i], 0, 0)),
            pl.BlockSpec((1, H, D), lambda i, te: (te[i], 0, 0)),
            pl.BlockSpec((1, D, H), lambda i, te: (te[i], 0, 0)),
        ],
        out_specs=pl.BlockSpec((TM2, D), lambda i, te: (i, 0)),
    )
    return pl.pallas_call(
        _group_gemm_body,
        grid_spec=grid_spec,
        out_shape=jax.ShapeDtypeStruct((BUF, D), jnp.bfloat16),
        compiler_params=pltpu.CompilerParams(
            dimension_semantics=("arbitrary",)),
    )(te, gx, gate_w, up_w, down_w)


def _gather_body(y_hbm, posq_hbm, y1g_hbm, y2g_hbm,
                 y1_v, y2_v, idxa_v, idxb_v, sem):
    cid = lax.axis_index("c")
    sid = lax.axis_index("s")
    wid = sid * 2 + cid
    for c in range(2):
        base = wid * RPW + c * 64
        pltpu.sync_copy(posq_hbm.at[wid, c, 0], idxa_v)
        pltpu.sync_copy(posq_hbm.at[wid, c, 1], idxb_v)
        da = pltpu.async_copy(y_hbm.at[idxa_v], y1_v, sem)
        db = pltpu.async_copy(y_hbm.at[idxb_v], y2_v, sem)
        da.wait()
        db.wait()
        pltpu.sync_copy(y1_v, y1g_hbm.at[pl.ds(base, 64)])
        pltpu.sync_copy(y2_v, y2g_hbm.at[pl.ds(base, 64)])


def _gather(y32, posq):
    mesh = plsc.VectorSubcoreMesh(core_axis_name="c", subcore_axis_name="s")
    return pl.kernel(
        _gather_body,
        out_type=[
            jax.ShapeDtypeStruct((N, D // 2), jnp.int32),
            jax.ShapeDtypeStruct((N, D // 2), jnp.int32),
        ],
        mesh=mesh,
        scratch_types=[
            pltpu.VMEM((64, D // 2), jnp.int32),
            pltpu.VMEM((64, D // 2), jnp.int32),
            pltpu.VMEM((64,), jnp.int32),
            pltpu.VMEM((64,), jnp.int32),
            pltpu.SemaphoreType.DMA,
        ],
    )(y32, posq)


def _combine_body(sh_ref, y1_ref, y2_ref, w_ref, out_ref):
    w1 = w_ref[:, 0:1]
    w2 = w_ref[:, 1:2]
    out_ref[...] = (sh_ref[...]
                    + w1 * y1_ref[...].astype(jnp.float32)
                    + w2 * y2_ref[...].astype(jnp.float32))


def _combine(shared, y1g, y2g, w_out):
    return pl.pallas_call(
        _combine_body,
        grid=(N // TMS,),
        in_specs=[
            pl.BlockSpec((TMS, D), lambda t: (t, 0)),
            pl.BlockSpec((TMS, D), lambda t: (t, 0)),
            pl.BlockSpec((TMS, D), lambda t: (t, 0)),
            pl.BlockSpec((TMS, 128), lambda t: (t, 0)),
        ],
        out_specs=pl.BlockSpec((TMS, D), lambda t: (t, 0)),
        out_shape=jax.ShapeDtypeStruct((N, D), jnp.float32),
        compiler_params=pltpu.CompilerParams(
            dimension_semantics=("parallel",)),
    )(shared, y1g, y2g, w_out)


@jax.jit
def kernel(x, router_w, router_bias, gate_w, up_w, down_w, sg_w, su_w, sd_w):
    flat = x.reshape(N, D)
    rb = router_bias.reshape(1, E)
    xb = flat.astype(jnp.bfloat16)

    pos_out, w_out, off_out = _router_meta(flat, router_w, rb)

    pos_kn = pos_out[:, :TOPK].T
    posq = pos_kn.reshape(TOPK, NW, 2, 64).transpose(1, 2, 0, 3)
    off = off_out[0, :E]
    tile_start = jnp.arange(NTILES, dtype=jnp.int32) * TM2
    te = jnp.sum((off[None, :] <= tile_start[:, None]).astype(jnp.int32),
                 axis=1) - 1

    xb32 = lax.bitcast_convert_type(xb.reshape(N, D // 2, 2), jnp.int32)
    gx32 = _dispatch(xb32, posq)
    gx = lax.bitcast_convert_type(gx32, jnp.bfloat16).reshape(BUF, D)
    shared = _shared_ffn(flat, sg_w, su_w, sd_w)
    y = _group_gemm(te, gx, gate_w, up_w, down_w)
    y32 = lax.bitcast_convert_type(y.reshape(BUF, D // 2, 2), jnp.int32)
    y1g32, y2g32 = _gather(y32, posq)
    y1g = lax.bitcast_convert_type(y1g32, jnp.bfloat16).reshape(N, D)
    y2g = lax.bitcast_convert_type(y2g32, jnp.bfloat16).reshape(N, D)
    out = _combine(shared, y1g, y2g, w_out)
    return out.reshape(B, T, D)


# speedup vs baseline: 4.4923x; 4.4923x over previous
import functools

import jax
import jax.numpy as jnp
from jax import lax
from jax.experimental import pallas as pl
from jax.experimental.pallas import tpu as pltpu
from jax.experimental.pallas import tpu_sc as plsc

B, T, D = 2, 2048, 1024
E, TOPK, H = 8, 2, 512
SH = H * TOPK
N = B * T
M = N * TOPK
TM2 = 256
BUF = M + E * TM2
NTILES = BUF // TM2
NW = 32
RPW = N // NW
TMS = 1024


def _cumsum_rows(a):
    rows = lax.broadcasted_iota(jnp.int32, a.shape, 0)
    k = 1
    while k < a.shape[0]:
        shifted = pltpu.roll(a, k, axis=0)
        a = a + jnp.where(rows >= k, shifted, jnp.zeros_like(a))
        k *= 2
    return a


def _router_body(x_ref, rw_ref, rb_ref, pos_ref, w_ref, off_ref):
    x = x_ref[...]
    logits = jnp.dot(x, rw_ref[...].T, preferred_element_type=jnp.float32)
    logits = logits + rb_ref[...]
    scores = jax.nn.softmax(logits, axis=-1)
    s1 = jnp.max(scores, axis=-1, keepdims=True)
    i1 = jnp.argmax(scores, axis=-1).reshape(N, 1)
    cols = lax.broadcasted_iota(jnp.int32, (N, E), 1)
    masked = jnp.where(cols == i1, -jnp.inf, scores)
    s2 = jnp.max(masked, axis=-1, keepdims=True)
    i2 = jnp.argmax(masked, axis=-1).reshape(N, 1)
    denom = s1 + s2
    w1 = s1 / denom
    w2 = s2 / denom

    oh1 = (cols == i1).astype(jnp.int32)
    oh2 = (cols == i2).astype(jnp.int32)
    packed = _cumsum_rows(oh1 + (oh2 << 13))
    cs1 = packed & 8191
    cs2 = packed >> 13
    cnt1 = cs1[N - 1:N, :]
    counts = cnt1 + cs2[N - 1:N, :]
    padded = (counts + (TM2 - 1)) & (-TM2)
    padded_f = padded.astype(jnp.float32)
    r8 = lax.broadcasted_iota(jnp.int32, (E, E), 0)
    c8 = lax.broadcasted_iota(jnp.int32, (E, E), 1)
    tril = (r8 < c8).astype(jnp.float32)
    offf = jnp.dot(padded_f, tril, preferred_element_type=jnp.float32)
    off = offf.astype(jnp.int32)

    rk1 = jnp.sum(oh1 * cs1, axis=1, keepdims=True) - 1
    rk2 = jnp.sum(oh2 * (cs2 + cnt1), axis=1, keepdims=True) - 1
    base1 = jnp.sum(oh1 * off, axis=1, keepdims=True)
    base2 = jnp.sum(oh2 * off, axis=1, keepdims=True)
    pos1 = base1 + rk1
    pos2 = base2 + rk2

    c128 = lax.broadcasted_iota(jnp.int32, (N, 128), 1)
    pos_ref[...] = jnp.where(c128 == 0, pos1, jnp.where(c128 == 1, pos2, 0))
    w_ref[...] = jnp.where(c128 == 0, w1, jnp.where(c128 == 1, w2, 0.0))
    spread = (lax.broadcasted_iota(jnp.int32, (E, 128), 0)
              == lax.broadcasted_iota(jnp.int32, (E, 128), 1)).astype(jnp.float32)
    off_ref[...] = jnp.dot(offf, spread,
                           preferred_element_type=jnp.float32).astype(jnp.int32)


def _router_meta(flat, router_w, rb):
    return pl.pallas_call(
        _router_body,
        out_shape=[
            jax.ShapeDtypeStruct((N, 128), jnp.int32),
            jax.ShapeDtypeStruct((N, 128), jnp.float32),
            jax.ShapeDtypeStruct((1, 128), jnp.int32),
        ],
    )(flat, router_w, rb)


def _shared_body(x_ref, sg_ref, su_ref, sd_ref, out_ref):
    x = x_ref[...]
    g = jnp.dot(x, sg_ref[...].T, preferred_element_type=jnp.float32)
    u = jnp.dot(x, su_ref[...].T, preferred_element_type=jnp.float32)
    h = (g * jax.nn.sigmoid(g)) * u
    out_ref[...] = jnp.dot(h, sd_ref[...].T, preferred_element_type=jnp.float32)


def _shared_ffn(flat, sg_w, su_w, sd_w):
    return pl.pallas_call(
        _shared_body,
        grid=(N // TMS,),
        in_specs=[
            pl.BlockSpec((TMS, D), lambda t: (t, 0)),
            pl.BlockSpec((SH, D), lambda t: (0, 0)),
            pl.BlockSpec((SH, D), lambda t: (0, 0)),
            pl.BlockSpec((D, SH), lambda t: (0, 0)),
        ],
        out_specs=pl.BlockSpec((TMS, D), lambda t: (t, 0)),
        out_shape=jax.ShapeDtypeStruct((N, D), jnp.float32),
        compiler_params=pltpu.CompilerParams(
            dimension_semantics=("parallel",)),
    )(flat, sg_w, su_w, sd_w)


def _dispatch_body(xb_hbm, posq_hbm, gx_hbm, rows_v, idx0_v, idx1_v, sem):
    cid = lax.axis_index("c")
    sid = lax.axis_index("s")
    wid = sid * 2 + cid
    for c in range(2):
        base = wid * RPW + c * 64
        pltpu.sync_copy(xb_hbm.at[pl.ds(base, 64)], rows_v)
        pltpu.sync_copy(posq_hbm.at[wid, c, 0], idx0_v)
        pltpu.sync_copy(posq_hbm.at[wid, c, 1], idx1_v)
        d0 = pltpu.async_copy(rows_v, gx_hbm.at[idx0_v], sem)
        d1 = pltpu.async_copy(rows_v, gx_hbm.at[idx1_v], sem)
        d0.wait()
        d1.wait()


def _dispatch(flat, posq):
    mesh = plsc.VectorSubcoreMesh(core_axis_name="c", subcore_axis_name="s")
    return pl.kernel(
        _dispatch_body,
        out_type=jax.ShapeDtypeStruct((BUF, D), jnp.float32),
        mesh=mesh,
        scratch_types=[
            pltpu.VMEM((64, D), jnp.float32),
            pltpu.VMEM((64,), jnp.int32),
            pltpu.VMEM((64,), jnp.int32),
            pltpu.SemaphoreType.DMA,
        ],
    )(flat, posq)


def _group_gemm_body(te_ref, gx_ref, gw_ref, uw_ref, dw_ref, y_ref):
    xg = gx_ref[...]
    g = jnp.dot(xg, gw_ref[0].T, preferred_element_type=jnp.float32)
    u = jnp.dot(xg, uw_ref[0].T, preferred_element_type=jnp.float32)
    h = (g * jax.nn.sigmoid(g)) * u
    p = jnp.dot(h, dw_ref[0].T, preferred_element_type=jnp.float32)
    y_ref[...] = p


def _group_gemm(te, gx, gate_w, up_w, down_w):
    grid_spec = pltpu.PrefetchScalarGridSpec(
        num_scalar_prefetch=1,
        grid=(NTILES,),
        in_specs=[
            pl.BlockSpec((TM2, D), lambda i, te: (i, 0)),
            pl.BlockSpec((1, H, D), lambda i, te: (te[i], 0, 0)),
            pl.BlockSpec((1, H, D), lambda i, te: (te[i], 0, 0)),
            pl.BlockSpec((1, D, H), lambda i, te: (te[i], 0, 0)),
        ],
        out_specs=pl.BlockSpec((TM2, D), lambda i, te: (i, 0)),
    )
    return pl.pallas_call(
        _group_gemm_body,
        grid_spec=grid_spec,
        out_shape=jax.ShapeDtypeStruct((BUF, D), jnp.float32),
        compiler_params=pltpu.CompilerParams(
            dimension_semantics=("arbitrary",)),
    )(te, gx, gate_w, up_w, down_w)


def _gather_body(y_hbm, posg_hbm, y1g_hbm, y2g_hbm,
                 y1_v, y2_v, idxa_v, idxb_v, sem):
    cid = lax.axis_index("c")
    sid = lax.axis_index("s")
    wid = sid * 2 + cid
    for c in range(4):
        base = wid * RPW + c * 32
        pltpu.sync_copy(posg_hbm.at[wid, c, 0], idxa_v)
        pltpu.sync_copy(posg_hbm.at[wid, c, 1], idxb_v)
        da = pltpu.async_copy(y_hbm.at[idxa_v], y1_v, sem)
        db = pltpu.async_copy(y_hbm.at[idxb_v], y2_v, sem)
        da.wait()
        db.wait()
        pltpu.sync_copy(y1_v, y1g_hbm.at[pl.ds(base, 32)])
        pltpu.sync_copy(y2_v, y2g_hbm.at[pl.ds(base, 32)])


def _gather(y, posg):
    mesh = plsc.VectorSubcoreMesh(core_axis_name="c", subcore_axis_name="s")
    return pl.kernel(
        _gather_body,
        out_type=[
            jax.ShapeDtypeStruct((N, D), jnp.float32),
            jax.ShapeDtypeStruct((N, D), jnp.float32),
        ],
        mesh=mesh,
        scratch_types=[
            pltpu.VMEM((32, D), jnp.float32),
            pltpu.VMEM((32, D), jnp.float32),
            pltpu.VMEM((32,), jnp.int32),
            pltpu.VMEM((32,), jnp.int32),
            pltpu.SemaphoreType.DMA,
        ],
    )(y, posg)


def _combine_body(sh_ref, y1_ref, y2_ref, w_ref, out_ref):
    w1 = w_ref[:, 0:1]
    w2 = w_ref[:, 1:2]
    out_ref[...] = sh_ref[...] + w1 * y1_ref[...] + w2 * y2_ref[...]


def _combine(shared, y1g, y2g, w_out):
    return pl.pallas_call(
        _combine_body,
        grid=(N // TMS,),
        in_specs=[
            pl.BlockSpec((TMS, D), lambda t: (t, 0)),
            pl.BlockSpec((TMS, D), lambda t: (t, 0)),
            pl.BlockSpec((TMS, D), lambda t: (t, 0)),
            pl.BlockSpec((TMS, 128), lambda t: (t, 0)),
        ],
        out_specs=pl.BlockSpec((TMS, D), lambda t: (t, 0)),
        out_shape=jax.ShapeDtypeStruct((N, D), jnp.float32),
        compiler_params=pltpu.CompilerParams(
            dimension_semantics=("parallel",)),
    )(shared, y1g, y2g, w_out)


@jax.jit
def kernel(x, router_w, router_bias, gate_w, up_w, down_w, sg_w, su_w, sd_w):
    flat = x.reshape(N, D)
    rb = router_bias.reshape(1, E)

    pos_out, w_out, off_out = _router_meta(flat, router_w, rb)

    pos_kn = pos_out[:, :TOPK].T
    posq = pos_kn.reshape(TOPK, NW, 2, 64).transpose(1, 2, 0, 3)
    posg = pos_kn.reshape(TOPK, NW, 4, 32).transpose(1, 2, 0, 3)
    off = off_out[0, :E]
    tile_start = jnp.arange(NTILES, dtype=jnp.int32) * TM2
    te = jnp.sum((off[None, :] <= tile_start[:, None]).astype(jnp.int32),
                 axis=1) - 1

    gx = _dispatch(flat, posq)
    shared = _shared_ffn(flat, sg_w, su_w, sd_w)
    y = _group_gemm(te, gx, gate_w, up_w, down_w)
    y1g, y2g = _gather(y, posg)
    out = _combine(shared, y1g, y2g, w_out)
    return out.reshape(B, T, D)


# speedup vs baseline: 4.9486x; 1.1016x over previous
import functools

import jax
import jax.numpy as jnp
from jax import lax
from jax.experimental import pallas as pl
from jax.experimental.pallas import tpu as pltpu
from jax.experimental.pallas import tpu_sc as plsc

B, T, D = 2, 2048, 1024
E, TOPK, H = 8, 2, 512
SH = H * TOPK
N = B * T
M = N * TOPK
TM2 = 256
BUF = M + E * TM2
NTILES = BUF // TM2
NW = 32
RPW = N // NW
TMS = 1024
DP = D // 2


def _cumsum_rows(a):
    rows = lax.broadcasted_iota(jnp.int32, a.shape, 0)
    k = 1
    while k < a.shape[0]:
        shifted = pltpu.roll(a, k, axis=0)
        a = a + jnp.where(rows >= k, shifted, jnp.zeros_like(a))
        k *= 2
    return a


def _pack_bf16(v):
    u = lax.bitcast_convert_type(v, jnp.uint32)
    half = v.shape[1] // 2
    rnd = lambda b: (b + jnp.uint32(0x7FFF) + ((b >> 16) & 1)) >> 16
    lo = rnd(u[:, :half]) & jnp.uint32(0xFFFF)
    hi = rnd(u[:, half:])
    return lax.bitcast_convert_type(lo | (hi << 16), jnp.int32)


def _unpack_bf16(p):
    u = lax.bitcast_convert_type(p, jnp.uint32)
    lo = lax.bitcast_convert_type((u & jnp.uint32(0xFFFF)) << 16, jnp.float32)
    hi = lax.bitcast_convert_type(u & jnp.uint32(0xFFFF0000), jnp.float32)
    return jnp.concatenate([lo, hi], axis=1)


def _router_body(x_ref, rw_ref, rb_ref, pos_ref, w_ref, off_ref, xpk_ref):
    x = x_ref[...]
    xpk_ref[...] = _pack_bf16(x)
    logits = jnp.dot(x, rw_ref[...].T, preferred_element_type=jnp.float32)
    logits = logits + rb_ref[...]
    scores = jax.nn.softmax(logits, axis=-1)
    s1 = jnp.max(scores, axis=-1, keepdims=True)
    i1 = jnp.argmax(scores, axis=-1).reshape(N, 1)
    cols = lax.broadcasted_iota(jnp.int32, (N, E), 1)
    masked = jnp.where(cols == i1, -jnp.inf, scores)
    s2 = jnp.max(masked, axis=-1, keepdims=True)
    i2 = jnp.argmax(masked, axis=-1).reshape(N, 1)
    denom = s1 + s2
    w1 = s1 / denom
    w2 = s2 / denom

    oh1 = (cols == i1).astype(jnp.int32)
    oh2 = (cols == i2).astype(jnp.int32)
    packed = _cumsum_rows(oh1 + (oh2 << 13))
    cs1 = packed & 8191
    cs2 = packed >> 13
    cnt1 = cs1[N - 1:N, :]
    counts = cnt1 + cs2[N - 1:N, :]
    padded = (counts + (TM2 - 1)) & (-TM2)
    padded_f = padded.astype(jnp.float32)
    r8 = lax.broadcasted_iota(jnp.int32, (E, E), 0)
    c8 = lax.broadcasted_iota(jnp.int32, (E, E), 1)
    tril = (r8 < c8).astype(jnp.float32)
    offf = jnp.dot(padded_f, tril, preferred_element_type=jnp.float32)
    off = offf.astype(jnp.int32)

    rk1 = jnp.sum(oh1 * cs1, axis=1, keepdims=True) - 1
    rk2 = jnp.sum(oh2 * (cs2 + cnt1), axis=1, keepdims=True) - 1
    base1 = jnp.sum(oh1 * off, axis=1, keepdims=True)
    base2 = jnp.sum(oh2 * off, axis=1, keepdims=True)
    pos1 = base1 + rk1
    pos2 = base2 + rk2

    c128 = lax.broadcasted_iota(jnp.int32, (N, 128), 1)
    pos_ref[...] = jnp.where(c128 == 0, pos1, jnp.where(c128 == 1, pos2, 0))
    w_ref[...] = jnp.where(c128 == 0, w1, jnp.where(c128 == 1, w2, 0.0))
    spread = (lax.broadcasted_iota(jnp.int32, (E, 128), 0)
              == lax.broadcasted_iota(jnp.int32, (E, 128), 1)).astype(jnp.float32)
    off_ref[...] = jnp.dot(offf, spread,
                           preferred_element_type=jnp.float32).astype(jnp.int32)


def _router_meta(flat, router_w, rb):
    return pl.pallas_call(
        _router_body,
        out_shape=[
            jax.ShapeDtypeStruct((N, 128), jnp.int32),
            jax.ShapeDtypeStruct((N, 128), jnp.float32),
            jax.ShapeDtypeStruct((1, 128), jnp.int32),
            jax.ShapeDtypeStruct((N, DP), jnp.int32),
        ],
    )(flat, router_w, rb)


def _shared_body(x_ref, sg_ref, su_ref, sd_ref, out_ref):
    x = x_ref[...]
    g = jnp.dot(x, sg_ref[...].T, preferred_element_type=jnp.float32)
    u = jnp.dot(x, su_ref[...].T, preferred_element_type=jnp.float32)
    h = (g * jax.nn.sigmoid(g)) * u
    out_ref[...] = jnp.dot(h, sd_ref[...].T, preferred_element_type=jnp.float32)


def _shared_ffn(flat, sg_w, su_w, sd_w):
    return pl.pallas_call(
        _shared_body,
        grid=(N // TMS,),
        in_specs=[
            pl.BlockSpec((TMS, D), lambda t: (t, 0)),
            pl.BlockSpec((SH, D), lambda t: (0, 0)),
            pl.BlockSpec((SH, D), lambda t: (0, 0)),
            pl.BlockSpec((D, SH), lambda t: (0, 0)),
        ],
        out_specs=pl.BlockSpec((TMS, D), lambda t: (t, 0)),
        out_shape=jax.ShapeDtypeStruct((N, D), jnp.float32),
        compiler_params=pltpu.CompilerParams(
            dimension_semantics=("parallel",)),
    )(flat, sg_w, su_w, sd_w)


def _dispatch_body(xpk_hbm, posq_hbm, gx_hbm, rows_v, idx0_v, idx1_v, sem):
    cid = lax.axis_index("c")
    sid = lax.axis_index("s")
    wid = sid * 2 + cid
    for c in range(2):
        base = wid * RPW + c * 64
        pltpu.sync_copy(xpk_hbm.at[pl.ds(base, 64)], rows_v)
        pltpu.sync_copy(posq_hbm.at[wid, c, 0], idx0_v)
        pltpu.sync_copy(posq_hbm.at[wid, c, 1], idx1_v)
        d0 = pltpu.async_copy(rows_v, gx_hbm.at[idx0_v], sem)
        d1 = pltpu.async_copy(rows_v, gx_hbm.at[idx1_v], sem)
        d0.wait()
        d1.wait()


def _dispatch(xpk, posq):
    mesh = plsc.VectorSubcoreMesh(core_axis_name="c", subcore_axis_name="s")
    return pl.kernel(
        _dispatch_body,
        out_type=jax.ShapeDtypeStruct((BUF, DP), jnp.int32),
        mesh=mesh,
        scratch_types=[
            pltpu.VMEM((64, DP), jnp.int32),
            pltpu.VMEM((64,), jnp.int32),
            pltpu.VMEM((64,), jnp.int32),
            pltpu.SemaphoreType.DMA,
        ],
    )(xpk, posq)


def _group_gemm_body(te_ref, gx_ref, gw_ref, uw_ref, dw_ref, y_ref):
    xg = _unpack_bf16(gx_ref[...])
    g = jnp.dot(xg, gw_ref[0].T, preferred_element_type=jnp.float32)
    u = jnp.dot(xg, uw_ref[0].T, preferred_element_type=jnp.float32)
    h = (g * jax.nn.sigmoid(g)) * u
    p = jnp.dot(h, dw_ref[0].T, preferred_element_type=jnp.float32)
    y_ref[...] = _pack_bf16(p)


def _group_gemm(te, gx, gate_w, up_w, down_w):
    grid_spec = pltpu.PrefetchScalarGridSpec(
        num_scalar_prefetch=1,
        grid=(NTILES,),
        in_specs=[
            pl.BlockSpec((TM2, DP), lambda i, te: (i, 0)),
            pl.BlockSpec((1, H, D), lambda i, te: (te[i], 0, 0)),
            pl.BlockSpec((1, H, D), lambda i, te: (te[i], 0, 0)),
            pl.BlockSpec((1, D, H), lambda i, te: (te[i], 0, 0)),
        ],
        out_specs=pl.BlockSpec((TM2, DP), lambda i, te: (i, 0)),
    )
    return pl.pallas_call(
        _group_gemm_body,
        grid_spec=grid_spec,
        out_shape=jax.ShapeDtypeStruct((BUF, DP), jnp.int32),
        compiler_params=pltpu.CompilerParams(
            dimension_semantics=("arbitrary",)),
    )(te, gx, gate_w, up_w, down_w)


def _gather_body(y_hbm, posg_hbm, y1g_hbm, y2g_hbm,
                 y1_v, y2_v, idxa_v, idxb_v, sem):
    cid = lax.axis_index("c")
    sid = lax.axis_index("s")
    wid = sid * 2 + cid
    for c in range(2):
        base = wid * RPW + c * 64
        pltpu.sync_copy(posg_hbm.at[wid, c, 0], idxa_v)
        pltpu.sync_copy(posg_hbm.at[wid, c, 1], idxb_v)
        da = pltpu.async_copy(y_hbm.at[idxa_v], y1_v, sem)
        db = pltpu.async_copy(y_hbm.at[idxb_v], y2_v, sem)
        da.wait()
        db.wait()
        pltpu.sync_copy(y1_v, y1g_hbm.at[pl.ds(base, 64)])
        pltpu.sync_copy(y2_v, y2g_hbm.at[pl.ds(base, 64)])


def _gather(y, posg):
    mesh = plsc.VectorSubcoreMesh(core_axis_name="c", subcore_axis_name="s")
    return pl.kernel(
        _gather_body,
        out_type=[
            jax.ShapeDtypeStruct((N, DP), jnp.int32),
            jax.ShapeDtypeStruct((N, DP), jnp.int32),
        ],
        mesh=mesh,
        scratch_types=[
            pltpu.VMEM((64, DP), jnp.int32),
            pltpu.VMEM((64, DP), jnp.int32),
            pltpu.VMEM((64,), jnp.int32),
            pltpu.VMEM((64,), jnp.int32),
            pltpu.SemaphoreType.DMA,
        ],
    )(y, posg)


def _combine_body(sh_ref, y1_ref, y2_ref, w_ref, out_ref):
    w1 = w_ref[:, 0:1]
    w2 = w_ref[:, 1:2]
    y1 = _unpack_bf16(y1_ref[...])
    y2 = _unpack_bf16(y2_ref[...])
    out_ref[...] = sh_ref[...] + w1 * y1 + w2 * y2


def _combine(shared, y1g, y2g, w_out):
    return pl.pallas_call(
        _combine_body,
        grid=(N // TMS,),
        in_specs=[
            pl.BlockSpec((TMS, D), lambda t: (t, 0)),
            pl.BlockSpec((TMS, DP), lambda t: (t, 0)),
            pl.BlockSpec((TMS, DP), lambda t: (t, 0)),
            pl.BlockSpec((TMS, 128), lambda t: (t, 0)),
        ],
        out_specs=pl.BlockSpec((TMS, D), lambda t: (t, 0)),
        out_shape=jax.ShapeDtypeStruct((N, D), jnp.float32),
        compiler_params=pltpu.CompilerParams(
            dimension_semantics=("parallel",)),
    )(shared, y1g, y2g, w_out)


@jax.jit
def kernel(x, router_w, router_bias, gate_w, up_w, down_w, sg_w, su_w, sd_w):
    flat = x.reshape(N, D)
    rb = router_bias.reshape(1, E)

    pos_out, w_out, off_out, xpk = _router_meta(flat, router_w, rb)

    pos_kn = pos_out[:, :TOPK].T
    posq = pos_kn.reshape(TOPK, NW, 2, 64).transpose(1, 2, 0, 3)
    off = off_out[0, :E]
    tile_start = jnp.arange(NTILES, dtype=jnp.int32) * TM2
    te = jnp.sum((off[None, :] <= tile_start[:, None]).astype(jnp.int32),
                 axis=1) - 1

    gx = _dispatch(xpk, posq)
    shared = _shared_ffn(flat, sg_w, su_w, sd_w)
    y = _group_gemm(te, gx, gate_w, up_w, down_w)
    y1g, y2g = _gather(y, posq)
    out = _combine(shared, y1g, y2g, w_out)
    return out.reshape(B, T, D)


# speedup vs baseline: 4.9960x; 1.0096x over previous
import functools

import jax
import jax.numpy as jnp
from jax import lax
from jax.experimental import pallas as pl
from jax.experimental.pallas import tpu as pltpu
from jax.experimental.pallas import tpu_sc as plsc

B, T, D = 2, 2048, 1024
E, TOPK, H = 8, 2, 512
SH = H * TOPK
N = B * T
M = N * TOPK
TM2 = 256
BUF = M + E * TM2
NTILES = BUF // TM2
NW = 32
RPW = N // NW
TMS = 1024
DP = D // 2


def _cumsum_rows(a):
    rows = lax.broadcasted_iota(jnp.int32, a.shape, 0)
    k = 1
    while k < a.shape[0]:
        shifted = pltpu.roll(a, k, axis=0)
        a = a + jnp.where(rows >= k, shifted, jnp.zeros_like(a))
        k *= 2
    return a


def _pack_bf16(v):
    u = lax.bitcast_convert_type(v, jnp.uint32)
    half = v.shape[1] // 2
    rnd = lambda b: (b + jnp.uint32(0x7FFF) + ((b >> 16) & 1)) >> 16
    lo = rnd(u[:, :half]) & jnp.uint32(0xFFFF)
    hi = rnd(u[:, half:])
    return lax.bitcast_convert_type(lo | (hi << 16), jnp.int32)


def _unpack_bf16(p):
    u = lax.bitcast_convert_type(p, jnp.uint32)
    lo = lax.bitcast_convert_type((u & jnp.uint32(0xFFFF)) << 16, jnp.float32)
    hi = lax.bitcast_convert_type(u & jnp.uint32(0xFFFF0000), jnp.float32)
    return jnp.concatenate([lo, hi], axis=1)


def _router_body(x_ref, rw_ref, rb_ref, pos_ref, w_ref, off_ref, xpk_ref):
    x = x_ref[...]
    xpk_ref[...] = _pack_bf16(x)
    logits = jnp.dot(x, rw_ref[...].T, preferred_element_type=jnp.float32)
    logits = logits + rb_ref[...]
    scores = jax.nn.softmax(logits, axis=-1)
    s1 = jnp.max(scores, axis=-1, keepdims=True)
    i1 = jnp.argmax(scores, axis=-1).reshape(N, 1)
    cols = lax.broadcasted_iota(jnp.int32, (N, E), 1)
    masked = jnp.where(cols == i1, -jnp.inf, scores)
    s2 = jnp.max(masked, axis=-1, keepdims=True)
    i2 = jnp.argmax(masked, axis=-1).reshape(N, 1)
    denom = s1 + s2
    w1 = s1 / denom
    w2 = s2 / denom

    oh1 = (cols == i1).astype(jnp.int32)
    oh2 = (cols == i2).astype(jnp.int32)
    packed = _cumsum_rows(oh1 + (oh2 << 13))
    cs1 = packed & 8191
    cs2 = packed >> 13
    cnt1 = cs1[N - 1:N, :]
    counts = cnt1 + cs2[N - 1:N, :]
    padded = (counts + (TM2 - 1)) & (-TM2)
    padded_f = padded.astype(jnp.float32)
    r8 = lax.broadcasted_iota(jnp.int32, (E, E), 0)
    c8 = lax.broadcasted_iota(jnp.int32, (E, E), 1)
    tril = (r8 < c8).astype(jnp.float32)
    offf = jnp.dot(padded_f, tril, preferred_element_type=jnp.float32)
    off = offf.astype(jnp.int32)

    rk1 = jnp.sum(oh1 * cs1, axis=1, keepdims=True) - 1
    rk2 = jnp.sum(oh2 * (cs2 + cnt1), axis=1, keepdims=True) - 1
    base1 = jnp.sum(oh1 * off, axis=1, keepdims=True)
    base2 = jnp.sum(oh2 * off, axis=1, keepdims=True)
    pos1 = base1 + rk1
    pos2 = base2 + rk2

    c128 = lax.broadcasted_iota(jnp.int32, (N, 128), 1)
    pos_ref[...] = jnp.where(c128 == 0, pos1, jnp.where(c128 == 1, pos2, 0))
    w_ref[...] = jnp.where(c128 == 0, w1, jnp.where(c128 == 1, w2, 0.0))
    spread = (lax.broadcasted_iota(jnp.int32, (E, 128), 0)
              == lax.broadcasted_iota(jnp.int32, (E, 128), 1)).astype(jnp.float32)
    off_ref[...] = jnp.dot(offf, spread,
                           preferred_element_type=jnp.float32).astype(jnp.int32)


def _router_meta(flat, router_w, rb):
    return pl.pallas_call(
        _router_body,
        out_shape=[
            jax.ShapeDtypeStruct((N, 128), jnp.int32),
            jax.ShapeDtypeStruct((N, 128), jnp.float32),
            jax.ShapeDtypeStruct((1, 128), jnp.int32),
            jax.ShapeDtypeStruct((N, DP), jnp.int32),
        ],
    )(flat, router_w, rb)


def _shared_body(x_ref, sg_ref, su_ref, sd_ref, out_ref):
    x = x_ref[...]
    g = jnp.dot(x, sg_ref[...].T, preferred_element_type=jnp.float32)
    u = jnp.dot(x, su_ref[...].T, preferred_element_type=jnp.float32)
    h = (g * jax.nn.sigmoid(g)) * u
    out_ref[...] = jnp.dot(h, sd_ref[...].T, preferred_element_type=jnp.float32)


def _shared_ffn(flat, sg_w, su_w, sd_w):
    return pl.pallas_call(
        _shared_body,
        grid=(N // TMS,),
        in_specs=[
            pl.BlockSpec((TMS, D), lambda t: (t, 0)),
            pl.BlockSpec((SH, D), lambda t: (0, 0)),
            pl.BlockSpec((SH, D), lambda t: (0, 0)),
            pl.BlockSpec((D, SH), lambda t: (0, 0)),
        ],
        out_specs=pl.BlockSpec((TMS, D), lambda t: (t, 0)),
        out_shape=jax.ShapeDtypeStruct((N, D), jnp.float32),
        compiler_params=pltpu.CompilerParams(
            dimension_semantics=("parallel",)),
    )(flat, sg_w, su_w, sd_w)


def _dispatch_body(xpk_hbm, posq_hbm, gx_hbm, rows_v, idx0_v, idx1_v, sem):
    cid = lax.axis_index("c")
    sid = lax.axis_index("s")
    wid = sid * 2 + cid
    for c in range(2):
        base = wid * RPW + c * 64
        pltpu.sync_copy(xpk_hbm.at[pl.ds(base, 64)], rows_v)
        pltpu.sync_copy(posq_hbm.at[wid, c, 0], idx0_v)
        pltpu.sync_copy(posq_hbm.at[wid, c, 1], idx1_v)
        d0 = pltpu.async_copy(rows_v, gx_hbm.at[idx0_v], sem)
        d1 = pltpu.async_copy(rows_v, gx_hbm.at[idx1_v], sem)
        d0.wait()
        d1.wait()


def _dispatch(xpk, posq):
    mesh = plsc.VectorSubcoreMesh(core_axis_name="c", subcore_axis_name="s")
    return pl.kernel(
        _dispatch_body,
        out_type=jax.ShapeDtypeStruct((BUF, DP), jnp.int32),
        mesh=mesh,
        scratch_types=[
            pltpu.VMEM((64, DP), jnp.int32),
            pltpu.VMEM((64,), jnp.int32),
            pltpu.VMEM((64,), jnp.int32),
            pltpu.SemaphoreType.DMA,
        ],
    )(xpk, posq)


def _group_gemm_body(te_ref, gx_ref, gw_ref, uw_ref, dw_ref, y_ref):
    tev = te_ref[pl.program_id(0)]
    xg = _unpack_bf16(gx_ref[...])
    g = jnp.dot(xg, gw_ref[tev].T, preferred_element_type=jnp.float32)
    u = jnp.dot(xg, uw_ref[tev].T, preferred_element_type=jnp.float32)
    h = (g * jax.nn.sigmoid(g)) * u
    p = jnp.dot(h, dw_ref[tev].T, preferred_element_type=jnp.float32)
    y_ref[...] = _pack_bf16(p)


def _group_gemm(te, gx, gate_w, up_w, down_w):
    grid_spec = pltpu.PrefetchScalarGridSpec(
        num_scalar_prefetch=1,
        grid=(NTILES,),
        in_specs=[
            pl.BlockSpec((TM2, DP), lambda i, te: (i, 0)),
            pl.BlockSpec((E, H, D), lambda i, te: (0, 0, 0)),
            pl.BlockSpec((E, H, D), lambda i, te: (0, 0, 0)),
            pl.BlockSpec((E, D, H), lambda i, te: (0, 0, 0)),
        ],
        out_specs=pl.BlockSpec((TM2, DP), lambda i, te: (i, 0)),
    )
    return pl.pallas_call(
        _group_gemm_body,
        grid_spec=grid_spec,
        out_shape=jax.ShapeDtypeStruct((BUF, DP), jnp.int32),
        compiler_params=pltpu.CompilerParams(
            dimension_semantics=("arbitrary",),
            vmem_limit_bytes=112 * 1024 * 1024),
    )(te, gx, gate_w, up_w, down_w)


def _gather_body(y_hbm, posg_hbm, y1g_hbm, y2g_hbm,
                 y1_v, y2_v, idxa_v, idxb_v, sem):
    cid = lax.axis_index("c")
    sid = lax.axis_index("s")
    wid = sid * 2 + cid
    for c in range(2):
        base = wid * RPW + c * 64
        pltpu.sync_copy(posg_hbm.at[wid, c, 0], idxa_v)
        pltpu.sync_copy(posg_hbm.at[wid, c, 1], idxb_v)
        da = pltpu.async_copy(y_hbm.at[idxa_v], y1_v, sem)
        db = pltpu.async_copy(y_hbm.at[idxb_v], y2_v, sem)
        da.wait()
        db.wait()
        pltpu.sync_copy(y1_v, y1g_hbm.at[pl.ds(base, 64)])
        pltpu.sync_copy(y2_v, y2g_hbm.at[pl.ds(base, 64)])


def _gather(y, posg):
    mesh = plsc.VectorSubcoreMesh(core_axis_name="c", subcore_axis_name="s")
    return pl.kernel(
        _gather_body,
        out_type=[
            jax.ShapeDtypeStruct((N, DP), jnp.int32),
            jax.ShapeDtypeStruct((N, DP), jnp.int32),
        ],
        mesh=mesh,
        scratch_types=[
            pltpu.VMEM((64, DP), jnp.int32),
            pltpu.VMEM((64, DP), jnp.int32),
            pltpu.VMEM((64,), jnp.int32),
            pltpu.VMEM((64,), jnp.int32),
            pltpu.SemaphoreType.DMA,
        ],
    )(y, posg)


def _combine_body(sh_ref, y1_ref, y2_ref, w_ref, out_ref):
    w1 = w_ref[:, 0:1]
    w2 = w_ref[:, 1:2]
    y1 = _unpack_bf16(y1_ref[...])
    y2 = _unpack_bf16(y2_ref[...])
    out_ref[...] = sh_ref[...] + w1 * y1 + w2 * y2


def _combine(shared, y1g, y2g, w_out):
    return pl.pallas_call(
        _combine_body,
        grid=(N // TMS,),
        in_specs=[
            pl.BlockSpec((TMS, D), lambda t: (t, 0)),
            pl.BlockSpec((TMS, DP), lambda t: (t, 0)),
            pl.BlockSpec((TMS, DP), lambda t: (t, 0)),
            pl.BlockSpec((TMS, 128), lambda t: (t, 0)),
        ],
        out_specs=pl.BlockSpec((TMS, D), lambda t: (t, 0)),
        out_shape=jax.ShapeDtypeStruct((N, D), jnp.float32),
        compiler_params=pltpu.CompilerParams(
            dimension_semantics=("parallel",)),
    )(shared, y1g, y2g, w_out)


@jax.jit
def kernel(x, router_w, router_bias, gate_w, up_w, down_w, sg_w, su_w, sd_w):
    flat = x.reshape(N, D)
    rb = router_bias.reshape(1, E)

    pos_out, w_out, off_out, xpk = _router_meta(flat, router_w, rb)

    pos_kn = pos_out[:, :TOPK].T
    posq = pos_kn.reshape(TOPK, NW, 2, 64).transpose(1, 2, 0, 3)
    off = off_out[0, :E]
    tile_start = jnp.arange(NTILES, dtype=jnp.int32) * TM2
    te = jnp.sum((off[None, :] <= tile_start[:, None]).astype(jnp.int32),
                 axis=1) - 1

    gx = _dispatch(xpk, posq)
    shared = _shared_ffn(flat, sg_w, su_w, sd_w)
    y = _group_gemm(te, gx, gate_w, up_w, down_w)
    y1g, y2g = _gather(y, posq)
    out = _combine(shared, y1g, y2g, w_out)
    return out.reshape(B, T, D)


# speedup vs baseline: 5.0417x; 1.0091x over previous
import functools

import jax
import jax.numpy as jnp
from jax import lax
from jax.experimental import pallas as pl
from jax.experimental.pallas import tpu as pltpu
from jax.experimental.pallas import tpu_sc as plsc

B, T, D = 2, 2048, 1024
E, TOPK, H = 8, 2, 512
SH = H * TOPK
N = B * T
M = N * TOPK
TM2 = 256
BUF = M + E * TM2
NTILES = BUF // TM2
NW = 32
RPW = N // NW
TMS = 1024
DP = D // 2


def _cumsum_rows(a):
    rows = lax.broadcasted_iota(jnp.int32, a.shape, 0)
    k = 1
    while k < a.shape[0]:
        shifted = pltpu.roll(a, k, axis=0)
        a = a + jnp.where(rows >= k, shifted, jnp.zeros_like(a))
        k *= 2
    return a


def _pack_bf16(v):
    u = lax.bitcast_convert_type(v, jnp.uint32)
    half = v.shape[1] // 2
    rnd = lambda b: (b + jnp.uint32(0x7FFF) + ((b >> 16) & 1)) >> 16
    lo = rnd(u[:, :half]) & jnp.uint32(0xFFFF)
    hi = rnd(u[:, half:])
    return lax.bitcast_convert_type(lo | (hi << 16), jnp.int32)


def _unpack_bf16(p):
    u = lax.bitcast_convert_type(p, jnp.uint32)
    lo = lax.bitcast_convert_type((u & jnp.uint32(0xFFFF)) << 16, jnp.float32)
    hi = lax.bitcast_convert_type(u & jnp.uint32(0xFFFF0000), jnp.float32)
    return jnp.concatenate([lo, hi], axis=1)


def _router_body(x_ref, rw_ref, rb_ref, pos_ref, w_ref, off_ref, xpk_ref):
    x = x_ref[...]
    xpk_ref[...] = _pack_bf16(x)
    logits = jnp.dot(x, rw_ref[...].T, preferred_element_type=jnp.float32)
    logits = logits + rb_ref[...]
    scores = jax.nn.softmax(logits, axis=-1)
    s1 = jnp.max(scores, axis=-1, keepdims=True)
    i1 = jnp.argmax(scores, axis=-1).reshape(N, 1)
    cols = lax.broadcasted_iota(jnp.int32, (N, E), 1)
    masked = jnp.where(cols == i1, -jnp.inf, scores)
    s2 = jnp.max(masked, axis=-1, keepdims=True)
    i2 = jnp.argmax(masked, axis=-1).reshape(N, 1)
    denom = s1 + s2
    w1 = s1 / denom
    w2 = s2 / denom

    oh1 = (cols == i1).astype(jnp.int32)
    oh2 = (cols == i2).astype(jnp.int32)
    packed = _cumsum_rows(oh1 + (oh2 << 13))
    cs1 = packed & 8191
    cs2 = packed >> 13
    cnt1 = cs1[N - 1:N, :]
    counts = cnt1 + cs2[N - 1:N, :]
    padded = (counts + (TM2 - 1)) & (-TM2)
    padded_f = padded.astype(jnp.float32)
    r8 = lax.broadcasted_iota(jnp.int32, (E, E), 0)
    c8 = lax.broadcasted_iota(jnp.int32, (E, E), 1)
    tril = (r8 < c8).astype(jnp.float32)
    offf = jnp.dot(padded_f, tril, preferred_element_type=jnp.float32)
    off = offf.astype(jnp.int32)

    rk1 = jnp.sum(oh1 * cs1, axis=1, keepdims=True) - 1
    rk2 = jnp.sum(oh2 * (cs2 + cnt1), axis=1, keepdims=True) - 1
    base1 = jnp.sum(oh1 * off, axis=1, keepdims=True)
    base2 = jnp.sum(oh2 * off, axis=1, keepdims=True)
    pos1 = base1 + rk1
    pos2 = base2 + rk2

    c128 = lax.broadcasted_iota(jnp.int32, (N, 128), 1)
    pos_ref[...] = jnp.where(c128 == 0, pos1, jnp.where(c128 == 1, pos2, 0))
    w_ref[...] = jnp.where(c128 == 0, w1, jnp.where(c128 == 1, w2, 0.0))
    spread = (lax.broadcasted_iota(jnp.int32, (E, 128), 0)
              == lax.broadcasted_iota(jnp.int32, (E, 128), 1)).astype(jnp.float32)
    off_ref[...] = jnp.dot(offf, spread,
                           preferred_element_type=jnp.float32).astype(jnp.int32)


def _router_meta(flat, router_w, rb):
    return pl.pallas_call(
        _router_body,
        out_shape=[
            jax.ShapeDtypeStruct((N, 128), jnp.int32),
            jax.ShapeDtypeStruct((N, 128), jnp.float32),
            jax.ShapeDtypeStruct((1, 128), jnp.int32),
            jax.ShapeDtypeStruct((N, DP), jnp.int32),
        ],
    )(flat, router_w, rb)


def _shared_body(x_ref, sg_ref, su_ref, sd_ref, out_ref):
    x = x_ref[...]
    g = jnp.dot(x, sg_ref[...].T, preferred_element_type=jnp.float32)
    u = jnp.dot(x, su_ref[...].T, preferred_element_type=jnp.float32)
    h = (g * jax.nn.sigmoid(g)) * u
    out_ref[...] = jnp.dot(h, sd_ref[...].T, preferred_element_type=jnp.float32)


def _shared_ffn(flat, sg_w, su_w, sd_w):
    return pl.pallas_call(
        _shared_body,
        grid=(N // TMS,),
        in_specs=[
            pl.BlockSpec((TMS, D), lambda t: (t, 0)),
            pl.BlockSpec((SH, D), lambda t: (0, 0)),
            pl.BlockSpec((SH, D), lambda t: (0, 0)),
            pl.BlockSpec((D, SH), lambda t: (0, 0)),
        ],
        out_specs=pl.BlockSpec((TMS, D), lambda t: (t, 0)),
        out_shape=jax.ShapeDtypeStruct((N, D), jnp.float32),
        compiler_params=pltpu.CompilerParams(
            dimension_semantics=("parallel",)),
    )(flat, sg_w, su_w, sd_w)


def _dispatch_body(xpk_hbm, posq_hbm, gx_hbm, rows_v, idx0_v, idx1_v, sem):
    cid = lax.axis_index("c")
    sid = lax.axis_index("s")
    wid = sid * 2 + cid
    for c in range(2):
        base = wid * RPW + c * 64
        pltpu.sync_copy(xpk_hbm.at[pl.ds(base, 64)], rows_v)
        pltpu.sync_copy(posq_hbm.at[wid, c, 0], idx0_v)
        pltpu.sync_copy(posq_hbm.at[wid, c, 1], idx1_v)
        d0 = pltpu.async_copy(rows_v, gx_hbm.at[idx0_v], sem)
        d1 = pltpu.async_copy(rows_v, gx_hbm.at[idx1_v], sem)
        d0.wait()
        d1.wait()


def _dispatch(xpk, posq):
    mesh = plsc.VectorSubcoreMesh(core_axis_name="c", subcore_axis_name="s")
    return pl.kernel(
        _dispatch_body,
        out_type=jax.ShapeDtypeStruct((BUF, DP), jnp.int32),
        mesh=mesh,
        scratch_types=[
            pltpu.VMEM((64, DP), jnp.int32),
            pltpu.VMEM((64,), jnp.int32),
            pltpu.VMEM((64,), jnp.int32),
            pltpu.SemaphoreType.DMA,
        ],
    )(xpk, posq)


def _group_gemm_body(te_ref, gx_ref, gw_ref, uw_ref, dw_ref, y_ref):
    tev = te_ref[pl.program_id(0)]
    xg = _unpack_bf16(gx_ref[...])
    g = jnp.dot(xg, gw_ref[tev].T, preferred_element_type=jnp.float32)
    u = jnp.dot(xg, uw_ref[tev].T, preferred_element_type=jnp.float32)
    h = (g * jax.nn.sigmoid(g)) * u
    p = jnp.dot(h, dw_ref[tev].T, preferred_element_type=jnp.float32)
    y_ref[...] = _pack_bf16(p)


def _group_gemm(te, gx, gate_w, up_w, down_w):
    grid_spec = pltpu.PrefetchScalarGridSpec(
        num_scalar_prefetch=1,
        grid=(NTILES,),
        in_specs=[
            pl.BlockSpec((TM2, DP), lambda i, te: (i, 0)),
            pl.BlockSpec((E, H, D), lambda i, te: (0, 0, 0)),
            pl.BlockSpec((E, H, D), lambda i, te: (0, 0, 0)),
            pl.BlockSpec((E, D, H), lambda i, te: (0, 0, 0)),
        ],
        out_specs=pl.BlockSpec((TM2, DP), lambda i, te: (i, 0)),
    )
    return pl.pallas_call(
        _group_gemm_body,
        grid_spec=grid_spec,
        out_shape=jax.ShapeDtypeStruct((BUF, DP), jnp.int32),
        compiler_params=pltpu.CompilerParams(
            dimension_semantics=("arbitrary",),
            vmem_limit_bytes=112 * 1024 * 1024),
    )(te, gx, gate_w, up_w, down_w)


def _gather_body(y_hbm, posg_hbm, y1g_hbm, y2g_hbm,
                 buf0, buf1, buf2, idx00, idx01, idx10, idx11,
                 sem0, sem1, sem2, sem3):
    cid = lax.axis_index("c")
    sid = lax.axis_index("s")
    wid = sid * 2 + cid
    base = wid * RPW
    pltpu.sync_copy(posg_hbm.at[wid, 0, 0], idx00)
    pltpu.sync_copy(posg_hbm.at[wid, 0, 1], idx01)
    pltpu.sync_copy(posg_hbm.at[wid, 1, 0], idx10)
    pltpu.sync_copy(posg_hbm.at[wid, 1, 1], idx11)
    d0 = pltpu.async_copy(y_hbm.at[idx00], buf0, sem0)
    d1 = pltpu.async_copy(y_hbm.at[idx01], buf1, sem1)
    d2 = pltpu.async_copy(y_hbm.at[idx10], buf2, sem2)
    d0.wait()
    pltpu.sync_copy(buf0, y1g_hbm.at[pl.ds(base, 64)])
    d3 = pltpu.async_copy(y_hbm.at[idx11], buf0, sem3)
    d1.wait()
    pltpu.sync_copy(buf1, y2g_hbm.at[pl.ds(base, 64)])
    d2.wait()
    pltpu.sync_copy(buf2, y1g_hbm.at[pl.ds(base + 64, 64)])
    d3.wait()
    pltpu.sync_copy(buf0, y2g_hbm.at[pl.ds(base + 64, 64)])


def _gather(y, posg):
    mesh = plsc.VectorSubcoreMesh(core_axis_name="c", subcore_axis_name="s")
    return pl.kernel(
        _gather_body,
        out_type=[
            jax.ShapeDtypeStruct((N, DP), jnp.int32),
            jax.ShapeDtypeStruct((N, DP), jnp.int32),
        ],
        mesh=mesh,
        scratch_types=[
            pltpu.VMEM((64, DP), jnp.int32),
            pltpu.VMEM((64, DP), jnp.int32),
            pltpu.VMEM((64, DP), jnp.int32),
            pltpu.VMEM((64,), jnp.int32),
            pltpu.VMEM((64,), jnp.int32),
            pltpu.VMEM((64,), jnp.int32),
            pltpu.VMEM((64,), jnp.int32),
            pltpu.SemaphoreType.DMA,
            pltpu.SemaphoreType.DMA,
            pltpu.SemaphoreType.DMA,
            pltpu.SemaphoreType.DMA,
        ],
    )(y, posg)


def _combine_body(sh_ref, y1_ref, y2_ref, w_ref, out_ref):
    w1 = w_ref[:, 0:1]
    w2 = w_ref[:, 1:2]
    y1 = _unpack_bf16(y1_ref[...])
    y2 = _unpack_bf16(y2_ref[...])
    out_ref[...] = sh_ref[...] + w1 * y1 + w2 * y2


def _combine(shared, y1g, y2g, w_out):
    return pl.pallas_call(
        _combine_body,
        grid=(N // TMS,),
        in_specs=[
            pl.BlockSpec((TMS, D), lambda t: (t, 0)),
            pl.BlockSpec((TMS, DP), lambda t: (t, 0)),
            pl.BlockSpec((TMS, DP), lambda t: (t, 0)),
            pl.BlockSpec((TMS, 128), lambda t: (t, 0)),
        ],
        out_specs=pl.BlockSpec((TMS, D), lambda t: (t, 0)),
        out_shape=jax.ShapeDtypeStruct((N, D), jnp.float32),
        compiler_params=pltpu.CompilerParams(
            dimension_semantics=("parallel",)),
    )(shared, y1g, y2g, w_out)


@jax.jit
def kernel(x, router_w, router_bias, gate_w, up_w, down_w, sg_w, su_w, sd_w):
    flat = x.reshape(N, D)
    rb = router_bias.reshape(1, E)

    pos_out, w_out, off_out, xpk = _router_meta(flat, router_w, rb)

    pos_kn = pos_out[:, :TOPK].T
    posq = pos_kn.reshape(TOPK, NW, 2, 64).transpose(1, 2, 0, 3)
    off = off_out[0, :E]
    tile_start = jnp.arange(NTILES, dtype=jnp.int32) * TM2
    te = jnp.sum((off[None, :] <= tile_start[:, None]).astype(jnp.int32),
                 axis=1) - 1

    gx = _dispatch(xpk, posq)
    shared = _shared_ffn(flat, sg_w, su_w, sd_w)
    y = _group_gemm(te, gx, gate_w, up_w, down_w)
    y1g, y2g = _gather(y, posq)
    out = _combine(shared, y1g, y2g, w_out)
    return out.reshape(B, T, D)
